# baseline (speedup 1.0000x reference)
import functools

import jax
import jax.numpy as jnp
from jax import lax
from jax.experimental import pallas as pl
from jax.experimental.pallas import tpu as pltpu
from jax.experimental.pallas import tpu_sc as plsc

VOCAB = 100000
HIDDEN = 1024
MAX_POS = 2048
B, S = 4, 2048
EPS = 1e-12

NW = 32
TOK_PER_W = (B * S) // NW
CHUNK = 32
NCHUNK = TOK_PER_W // CHUNK
LANES = 16
JV = HIDDEN // LANES


def _lane_sum(x):
    iota = lax.iota(jnp.int32, LANES)
    dnums = lax.GatherDimensionNumbers(
        offset_dims=(), collapsed_slice_dims=(0,), start_index_map=(0,))
    for shift in (1, 2, 4, 8):
        idx = (iota + shift) & (LANES - 1)
        perm = lax.gather(x, idx[:, None], dnums, (1,),
                          mode=lax.GatherScatterMode.PROMISE_IN_BOUNDS)
        x = x + perm
    return x


def _rsqrt_vec(x):
    y = 1.0 / (x + 0.1)
    half = x * 0.5
    for _ in range(10):
        y = y * (1.5 - half * y * y)
    return y


def _make_kernel():
    mesh = plsc.VectorSubcoreMesh(core_axis_name="c", subcore_axis_name="s")

    @functools.partial(
        pl.kernel,
        mesh=mesh,
        out_type=jax.ShapeDtypeStruct((B * S, HIDDEN), jnp.float32),
        scratch_types=[
            pltpu.VMEM((CHUNK,), jnp.int32),
            pltpu.VMEM((CHUNK,), jnp.int32),
            pltpu.VMEM((CHUNK, HIDDEN), jnp.float32),
            pltpu.VMEM((CHUNK, HIDDEN), jnp.float32),
            pltpu.VMEM((CHUNK, HIDDEN), jnp.float32),
            pltpu.VMEM((HIDDEN,), jnp.float32),
            pltpu.VMEM((HIDDEN,), jnp.float32),
            pltpu.SemaphoreType.DMA,
            pltpu.SemaphoreType.DMA,
        ],
    )
    def k(ids_hbm, tt_hbm, tok_hbm, pos_hbm, seg_hbm, gamma_hbm, beta_hbm,
          out_hbm, idx_v, tti_v, buf, pbuf, sbuf, gamma_v, beta_v, sem0, sem1):
        wid = lax.axis_index("c") * 16 + lax.axis_index("s")
        base = wid * TOK_PER_W

        pltpu.sync_copy(gamma_hbm, gamma_v)
        pltpu.sync_copy(beta_hbm, beta_v)

        def chunk_body(kk, _c):
            g0 = base + kk * CHUNK
            s0 = lax.rem(g0, S)

            pltpu.sync_copy(ids_hbm.at[pl.ds(g0, CHUNK)], idx_v)
            pltpu.sync_copy(tt_hbm.at[pl.ds(g0, CHUNK)], tti_v)
            cp0 = pltpu.async_copy(tok_hbm.at[idx_v], buf, sem0)
            cp1 = pltpu.async_copy(seg_hbm.at[tti_v], sbuf, sem1)
            pltpu.sync_copy(pos_hbm.at[pl.ds(s0, CHUNK)], pbuf)
            cp0.wait()
            cp1.wait()

            def body(t, _):
                a0 = jnp.zeros((LANES,), jnp.float32)
                a1 = jnp.zeros((LANES,), jnp.float32)
                q0 = jnp.zeros((LANES,), jnp.float32)
                q1 = jnp.zeros((LANES,), jnp.float32)
                for j in range(0, JV, 2):
                    sl0 = pl.ds((j + 0) * LANES, LANES)
                    sl1 = pl.ds((j + 1) * LANES, LANES)
                    v0 = buf[t, sl0] + pbuf[t, sl0] + sbuf[t, sl0]
                    v1 = buf[t, sl1] + pbuf[t, sl1] + sbuf[t, sl1]
                    buf[t, sl0] = v0
                    buf[t, sl1] = v1
                    a0 = a0 + v0
                    a1 = a1 + v1
                    q0 = q0 + v0 * v0
                    q1 = q1 + v1 * v1
                ssum_v = _lane_sum(a0 + a1)
                qsum_v = _lane_sum(q0 + q1)
                mean_v = ssum_v * (1.0 / HIDDEN)
                var_v = qsum_v * (1.0 / HIDDEN) - mean_v * mean_v
                rstd_v = _rsqrt_vec(var_v + EPS)
                for j in range(JV):
                    sl = pl.ds(j * LANES, LANES)
                    y = (buf[t, sl] - mean_v) * rstd_v * gamma_v[sl] + beta_v[sl]
                    buf[t, sl] = y
                return _

            lax.fori_loop(0, CHUNK, body, 0)

            pltpu.sync_copy(buf, out_hbm.at[pl.ds(g0, CHUNK)])
            return _c

        lax.fori_loop(0, NCHUNK, chunk_body, 0)

    return k


_kernel_call = _make_kernel()


def kernel(input_ids, token_type_ids, token_table, pos_table, seg_table,
           gamma, beta):
    ids = input_ids.reshape(-1).astype(jnp.int32)
    tt = token_type_ids.reshape(-1).astype(jnp.int32)
    out = _kernel_call(ids, tt, token_table, pos_table, seg_table, gamma, beta)
    return out.reshape(B, S, HIDDEN)
